# baseline (speedup 1.0000x reference)
import functools

import jax
import jax.numpy as jnp
from jax import lax
from jax.experimental import pallas as pl
from jax.experimental.pallas import tpu as pltpu
from jax.experimental.pallas import tpu_sc as plsc

_info = plsc.get_sparse_core_info()
_NC, _NS = _info.num_cores, _info.num_subcores
_NW = _NC * _NS

_N = 4 * 8192
_D = 1024
_TPW = _N // _NW
_K = 32
_NCHUNK = _TPW // _K


def _sc_body(ids_hbm, table_hbm, out_hbm, idx_v, buf0, buf1, gs0, gs1,
             ws0, ws1):
    wid = lax.axis_index("s") * _NC + lax.axis_index("c")
    base = wid * _TPW

    pltpu.sync_copy(ids_hbm.at[pl.ds(base, _TPW)], idx_v)

    bufs = (buf0, buf1)
    gsems = (gs0, gs1)
    wsems = (ws0, ws1)

    def start_gather(c, b):
        pltpu.async_copy(
            table_hbm.at[idx_v.at[pl.ds(c * _K, _K)]], bufs[b], gsems[b])

    def wait_gather(c, b):
        pltpu.make_async_copy(
            table_hbm.at[idx_v.at[pl.ds(c * _K, _K)]], bufs[b],
            gsems[b]).wait()

    def start_write(c, b):
        pltpu.async_copy(
            bufs[b], out_hbm.at[pl.ds(base + c * _K, _K)], wsems[b])

    def wait_write(c, b):
        pltpu.make_async_copy(
            bufs[b], out_hbm.at[pl.ds(base + c * _K, _K)], wsems[b]).wait()

    start_gather(0, 0)
    start_gather(1, 1)

    def body(p, carry):
        g0 = 2 * p
        wait_gather(g0, 0)
        start_write(g0, 0)
        wait_write(g0, 0)
        start_gather(g0 + 2, 0)
        wait_gather(g0 + 1, 1)
        start_write(g0 + 1, 1)
        wait_write(g0 + 1, 1)
        start_gather(g0 + 3, 1)
        return carry

    lax.fori_loop(0, _NCHUNK // 2 - 1, body, 0)

    last = _NCHUNK - 2
    wait_gather(last, 0)
    start_write(last, 0)
    wait_gather(last + 1, 1)
    start_write(last + 1, 1)
    wait_write(last, 0)
    wait_write(last + 1, 1)


@jax.jit
def _lookup(ids_flat, table):
    mesh = plsc.VectorSubcoreMesh(core_axis_name="c", subcore_axis_name="s")
    f = functools.partial(
        pl.kernel,
        out_type=jax.ShapeDtypeStruct((_N, _D), jnp.float32),
        mesh=mesh,
        scratch_types=[
            pltpu.VMEM((_TPW,), jnp.int32),
            pltpu.VMEM((_K, _D), jnp.float32),
            pltpu.VMEM((_K, _D), jnp.float32),
            pltpu.SemaphoreType.DMA,
            pltpu.SemaphoreType.DMA,
            pltpu.SemaphoreType.DMA,
            pltpu.SemaphoreType.DMA,
        ],
    )(_sc_body)
    return f(ids_flat, table)


def kernel(token_type_ids, embedding_weight):
    ids_flat = token_type_ids.astype(jnp.int32).reshape(_N)
    out = _lookup(ids_flat, embedding_weight)
    return out.reshape(token_type_ids.shape + (_D,))


# speedup vs baseline: 11.5166x; 11.5166x over previous
import functools

import jax
import jax.numpy as jnp
from jax import lax
from jax.experimental import pallas as pl
from jax.experimental.pallas import tpu as pltpu
from jax.experimental.pallas import tpu_sc as plsc

_info = plsc.get_sparse_core_info()
_NC, _NS = _info.num_cores, _info.num_subcores
_NW = _NC * _NS
_L = 16

_N = 4 * 8192
_D = 1024
_NG = _D // _L
_TPW = _N // _NW
_K = 32
_NCHUNK = _TPW // _K


def _sc_body(ids_hbm, table_hbm, out_hbm, idx_v, tbl_v, buf0, buf1, ws0,
             ws1):
    wid = lax.axis_index("s") * _NC + lax.axis_index("c")
    base = wid * _TPW

    pltpu.sync_copy(table_hbm, tbl_v)
    pltpu.sync_copy(ids_hbm.at[pl.ds(base, _TPW)], idx_v)

    bufs = (buf0, buf1)
    wsems = (ws0, ws1)

    def materialize(c, b):
        buf = bufs[b]
        scales = []
        for g in range(_K // _L):
            ids_vec = idx_v[pl.ds(c * _K + g * _L, _L)]
            for t in range(_L):
                s_i = jnp.full((_L,), ids_vec[t], jnp.int32)
                scales.append(s_i.astype(jnp.float32))

        def col_group(j, carry):
            w0 = tbl_v[0, pl.ds(j * _L, _L)]
            d = tbl_v[1, pl.ds(j * _L, _L)] - w0
            for k in range(_K):
                buf[k, pl.ds(j * _L, _L)] = w0 + scales[k] * d
            return carry

        lax.fori_loop(0, _NG, col_group, 0)

    def start_write(c, b):
        pltpu.async_copy(
            bufs[b], out_hbm.at[pl.ds(base + c * _K, _K)], wsems[b])

    def wait_write(c, b):
        pltpu.make_async_copy(
            bufs[b], out_hbm.at[pl.ds(base + c * _K, _K)], wsems[b]).wait()

    materialize(0, 0)
    start_write(0, 0)
    materialize(1, 1)
    start_write(1, 1)

    def body(p, carry):
        c0 = 2 * p + 2
        wait_write(c0 - 2, 0)
        materialize(c0, 0)
        start_write(c0, 0)
        wait_write(c0 - 1, 1)
        materialize(c0 + 1, 1)
        start_write(c0 + 1, 1)
        return carry

    lax.fori_loop(0, _NCHUNK // 2 - 1, body, 0)

    wait_write(_NCHUNK - 2, 0)
    wait_write(_NCHUNK - 1, 1)


@jax.jit
def _lookup(ids_flat, table):
    mesh = plsc.VectorSubcoreMesh(core_axis_name="c", subcore_axis_name="s")
    f = functools.partial(
        pl.kernel,
        out_type=jax.ShapeDtypeStruct((_N, _D), jnp.float32),
        mesh=mesh,
        scratch_types=[
            pltpu.VMEM((_TPW,), jnp.int32),
            pltpu.VMEM((2, _D), jnp.float32),
            pltpu.VMEM((_K, _D), jnp.float32),
            pltpu.VMEM((_K, _D), jnp.float32),
            pltpu.SemaphoreType.DMA,
            pltpu.SemaphoreType.DMA,
        ],
    )(_sc_body)
    return f(ids_flat, table)


def kernel(token_type_ids, embedding_weight):
    ids_flat = token_type_ids.astype(jnp.int32).reshape(_N)
    out = _lookup(ids_flat, embedding_weight)
    return out.reshape(token_type_ids.shape + (_D,))
